# Initial kernel scaffold; baseline (speedup 1.0000x reference)
import jax
import jax.numpy as jnp
from jax.experimental import pallas as pl


def kernel(x, edge_index, W0, b0, W1, b1, W2, b2):
    raise NotImplementedError("write your pallas kernel here")


# speedup vs baseline: 2.5049x; 2.5049x over previous
import functools

import jax
import jax.numpy as jnp
from jax.experimental import pallas as pl
from jax.experimental.pallas import tpu as pltpu

N = 10000
E = 160000
NPAD = 10240
BLK = 256
GRID = NPAD // BLK


def _isq(deg_blk):
    return jax.lax.rsqrt(deg_blk + 1.0)


def _elu(v):
    return jnp.where(v > 0, v, jnp.exp(jnp.minimum(v, 0.0)) - 1.0)


def _k1_body(x_ref, deg_ref, xs_ref):
    xs_ref[...] = x_ref[...] * _isq(deg_ref[...])


def _tc_scale(x_pad, deg_pad):
    return pl.pallas_call(
        _k1_body,
        grid=(GRID,),
        in_specs=[
            pl.BlockSpec((BLK, 256), lambda i: (i, 0)),
            pl.BlockSpec((BLK, 1), lambda i: (i, 0)),
        ],
        out_specs=pl.BlockSpec((BLK, 256), lambda i: (i, 0)),
        out_shape=jax.ShapeDtypeStruct((NPAD, 256), jnp.float32),
    )(x_pad, deg_pad)


def _k2_body(s_ref, xs_ref, deg_ref, w_ref, b_ref, outa_ref, outb_ref):
    isq = _isq(deg_ref[...])
    p = (s_ref[...] + xs_ref[...]) * isq
    h = _elu(jnp.dot(p, w_ref[...], preferred_element_type=jnp.float32)
             + b_ref[...])
    xs1 = h * isq
    outa_ref[...] = xs1[:, :256]
    outb_ref[...] = xs1[:, 256:]


def _tc_layer0(s0, xs0, deg_pad, W0, b0):
    return pl.pallas_call(
        _k2_body,
        grid=(GRID,),
        in_specs=[
            pl.BlockSpec((BLK, 256), lambda i: (i, 0)),
            pl.BlockSpec((BLK, 256), lambda i: (i, 0)),
            pl.BlockSpec((BLK, 1), lambda i: (i, 0)),
            pl.BlockSpec((256, 512), lambda i: (0, 0)),
            pl.BlockSpec((1, 512), lambda i: (0, 0)),
        ],
        out_specs=[
            pl.BlockSpec((BLK, 256), lambda i: (i, 0)),
            pl.BlockSpec((BLK, 256), lambda i: (i, 0)),
        ],
        out_shape=[
            jax.ShapeDtypeStruct((NPAD, 256), jnp.float32),
            jax.ShapeDtypeStruct((NPAD, 256), jnp.float32),
        ],
    )(s0, xs0, deg_pad, W0, b0)


def _k3_body(sa_ref, sb_ref, xa_ref, xb_ref, deg_ref, w1_ref, b1_ref, w2_ref,
             out_ref):
    isq = _isq(deg_ref[...])
    pa = (sa_ref[...] + xa_ref[...]) * isq
    pb = (sb_ref[...] + xb_ref[...]) * isq
    p = jnp.concatenate([pa, pb], axis=1)
    h = _elu(jnp.dot(p, w1_ref[...], preferred_element_type=jnp.float32)
             + b1_ref[...])
    y = jnp.dot(h, w2_ref[...], preferred_element_type=jnp.float32)
    out_ref[...] = y * isq


def _tc_layer1(s1a, s1b, xs1a, xs1b, deg_pad, W1, b1, W2):
    return pl.pallas_call(
        _k3_body,
        grid=(GRID,),
        in_specs=[
            pl.BlockSpec((BLK, 256), lambda i: (i, 0)),
            pl.BlockSpec((BLK, 256), lambda i: (i, 0)),
            pl.BlockSpec((BLK, 256), lambda i: (i, 0)),
            pl.BlockSpec((BLK, 256), lambda i: (i, 0)),
            pl.BlockSpec((BLK, 1), lambda i: (i, 0)),
            pl.BlockSpec((512, 512), lambda i: (0, 0)),
            pl.BlockSpec((1, 512), lambda i: (0, 0)),
            pl.BlockSpec((512, 256), lambda i: (0, 0)),
        ],
        out_specs=pl.BlockSpec((BLK, 256), lambda i: (i, 0)),
        out_shape=jax.ShapeDtypeStruct((NPAD, 256), jnp.float32),
    )(s1a, s1b, xs1a, xs1b, deg_pad, W1, b1, W2)


def _k4_body(s_ref, xs_ref, deg_ref, b_ref, out_ref):
    isq = _isq(deg_ref[...])
    out_ref[...] = (s_ref[...] + xs_ref[...]) * isq + b_ref[...]


def _tc_layer2(s2, xs2, deg_pad, b2):
    return pl.pallas_call(
        _k4_body,
        grid=(GRID,),
        in_specs=[
            pl.BlockSpec((BLK, 256), lambda i: (i, 0)),
            pl.BlockSpec((BLK, 256), lambda i: (i, 0)),
            pl.BlockSpec((BLK, 1), lambda i: (i, 0)),
            pl.BlockSpec((1, 256), lambda i: (0, 0)),
        ],
        out_specs=pl.BlockSpec((BLK, 256), lambda i: (i, 0)),
        out_shape=jax.ShapeDtypeStruct((NPAD, 256), jnp.float32),
    )(s2, xs2, deg_pad, b2)


def kernel(x, edge_index, W0, b0, W1, b1, W2, b2):
    src = edge_index[0]
    dst = edge_index[1]

    deg = jnp.zeros((NPAD,), jnp.float32).at[dst].add(1.0)
    deg_pad = deg.reshape(NPAD, 1)

    x_pad = jnp.zeros((NPAD, 256), jnp.float32).at[:N].set(x)

    def prop(vs):
        return jnp.zeros((NPAD, vs.shape[1]), jnp.float32).at[dst].add(vs[src])

    xs0 = _tc_scale(x_pad, deg_pad)
    s0 = prop(xs0)
    xs1a, xs1b = _tc_layer0(s0, xs0, deg_pad, W0, b0.reshape(1, 512))
    s1a = prop(xs1a)
    s1b = prop(xs1b)
    xs2 = _tc_layer1(s1a, s1b, xs1a, xs1b, deg_pad, W1, b1.reshape(1, 512), W2)
    s2 = prop(xs2)
    out = _tc_layer2(s2, xs2, deg_pad, b2.reshape(1, 256))
    return out[:N]


# speedup vs baseline: 4.3423x; 1.7335x over previous
import functools

import jax
import jax.numpy as jnp
from jax import lax
from jax.experimental import pallas as pl
from jax.experimental.pallas import tpu as pltpu
from jax.experimental.pallas import tpu_sc as plsc

N = 10000
E = 160000
NPAD = 10240
BLK = 256
GRID = NPAD // BLK

NW = 32
NPW = NPAD // NW
Q = 128
C = 2000
NVR = C // 16
CAP = E + Q
STAGE = C + Q + 16

_MESH = plsc.VectorSubcoreMesh(core_axis_name="c", subcore_axis_name="s")


def _plan_body(src_hbm, dst_hbm, slist, dlist, counts, deg_hbm,
               sbuf, dbuf, ss, sd, deg16, degout, cntv):
    wid = lax.axis_index("c") * 16 + lax.axis_index("s")
    lo = wid * NPW
    lanes = lax.iota(jnp.int32, 16)
    zeros16f = jnp.zeros((16,), jnp.float32)
    ones16f = jnp.ones((16,), jnp.float32)

    def _z(t, _):
        deg16[pl.ds(t * 16, 16)] = zeros16f
        return _
    lax.fori_loop(0, NPW, _z, 0)

    def _chunk(ci, carry):
        nflushed, rem = carry
        coff = pl.multiple_of(ci * C, 8)
        pltpu.sync_copy(src_hbm.at[pl.ds(coff, C)], sbuf)
        pltpu.sync_copy(dst_hbm.at[pl.ds(coff, C)], dbuf)

        def _vreg(j, rem):
            s = sbuf[pl.ds(j * 16, 16)]
            d = dbuf[pl.ds(j * 16, 16)]
            dl = d - lo
            m = (dl >= 0) & (dl < NPW)
            dlc = jnp.clip(dl, 0, NPW - 1)
            plsc.store_compressed(ss.at[pl.ds(rem, 16)], s, mask=m)
            plsc.store_compressed(sd.at[pl.ds(rem, 16)], dlc, mask=m)
            plsc.addupdate_scatter(deg16, [lanes * NPW + dlc], ones16f, mask=m)
            return rem + jnp.sum(m.astype(jnp.int32))
        rem = lax.fori_loop(0, NVR, _vreg, rem)

        nq = rem >> 7
        r0 = nq << 7

        def _flush(k, _):
            off = pl.multiple_of(wid * CAP + nflushed + (k << 7), 8)
            k7 = pl.multiple_of(k << 7, 8)
            pltpu.sync_copy(ss.at[pl.ds(k7, Q)], slist.at[pl.ds(off, Q)])
            pltpu.sync_copy(sd.at[pl.ds(k7, Q)], dlist.at[pl.ds(off, Q)])
            return _
        lax.fori_loop(0, nq, _flush, 0)
        for j in range(Q // 16):
            vs = ss[pl.ds(r0 + j * 16, 16)]
            vd = sd[pl.ds(r0 + j * 16, 16)]
            ss[pl.ds(j * 16, 16)] = vs
            sd[pl.ds(j * 16, 16)] = vd
        return nflushed + r0, rem - r0

    nflushed, rem = lax.fori_loop(0, E // C, _chunk, (0, 0))

    for j in range(Q // 16):
        idx = lanes + (j * 16)
        vs = ss[pl.ds(j * 16, 16)]
        vd = sd[pl.ds(j * 16, 16)]
        keep = idx < rem
        ss[pl.ds(j * 16, 16)] = jnp.where(keep, vs, 0)
        sd[pl.ds(j * 16, 16)] = jnp.where(keep, vd, NPW)

    @pl.when(rem > 0)
    def _():
        off = pl.multiple_of(wid * CAP + nflushed, 8)
        pltpu.sync_copy(ss.at[pl.ds(0, Q)], slist.at[pl.ds(off, Q)])
        pltpu.sync_copy(sd.at[pl.ds(0, Q)], dlist.at[pl.ds(off, Q)])

    cntv[...] = jnp.full((16,), nflushed + rem, jnp.int32)
    pltpu.sync_copy(cntv, counts.at[wid])

    def _red(t, _):
        acc = zeros16f
        for l in range(16):
            acc = acc + deg16[pl.ds(l * NPW + t * 16, 16)]
        degout[pl.ds(t * 16, 16)] = acc
        return _
    lax.fori_loop(0, NPW // 16, _red, 0)
    pltpu.sync_copy(degout, deg_hbm.at[pl.ds(pl.multiple_of(wid * NPW, 8), NPW)])


@functools.partial(
    pl.kernel,
    out_type=[
        jax.ShapeDtypeStruct((NW * CAP,), jnp.int32),
        jax.ShapeDtypeStruct((NW * CAP,), jnp.int32),
        jax.ShapeDtypeStruct((NW, 16), jnp.int32),
        jax.ShapeDtypeStruct((NPAD,), jnp.float32),
    ],
    mesh=_MESH,
    compiler_params=pltpu.CompilerParams(needs_layout_passes=False),
    scratch_types=[
        pltpu.VMEM((C,), jnp.int32),
        pltpu.VMEM((C,), jnp.int32),
        pltpu.VMEM((STAGE,), jnp.int32),
        pltpu.VMEM((STAGE,), jnp.int32),
        pltpu.VMEM((16 * NPW,), jnp.float32),
        pltpu.VMEM((NPW,), jnp.float32),
        pltpu.VMEM((16,), jnp.int32),
    ],
)
def _sc_plan(src_hbm, dst_hbm, slist, dlist, counts, deg_hbm,
             sbuf, dbuf, ss, sd, deg16, degout, cntv):
    _plan_body(src_hbm, dst_hbm, slist, dlist, counts, deg_hbm,
               sbuf, dbuf, ss, sd, deg16, degout, cntv)


ACCW = (NPW + 1) * 256


def _prop_body(xs_hbm, slist, dlist, counts, zrows, out_hbm,
               acc, gbuf, sidx, didx, cntv, sem):
    wid = lax.axis_index("c") * 16 + lax.axis_index("s")
    lanes = lax.iota(jnp.int32, 16)
    pltpu.sync_copy(counts.at[wid], cntv)
    cnt = jnp.max(cntv[...])
    nq = (cnt + (Q - 1)) >> 7
    pltpu.sync_copy(zrows, acc)

    def _quantum(k, _):
        off = pl.multiple_of(wid * CAP + (k << 7), 8)
        pltpu.sync_copy(slist.at[pl.ds(off, Q)], sidx)
        pltpu.sync_copy(dlist.at[pl.ds(off, Q)], didx)
        pltpu.async_copy(xs_hbm.at[sidx], gbuf, sem).wait()

        def _vreg(v, _):
            base_v = didx[pl.ds(v * 16, 16)] << 8
            for l in range(16):
                b = pl.multiple_of(
                    jnp.sum(jnp.where(lanes == l, base_v, 0)), 8)
                r = v * 16 + l
                for j in range(16):
                    val = gbuf[r, pl.ds(j * 16, 16)]
                    plsc.addupdate(acc.at[pl.ds(b + j * 16, 16)], val)
            return _
        lax.fori_loop(0, Q // 16, _vreg, 0)
        return _
    lax.fori_loop(0, nq, _quantum, 0)

    pltpu.sync_copy(
        acc.at[pl.ds(0, NPW * 256)],
        out_hbm.at[pl.ds(pl.multiple_of(wid * NPW * 256, 8), NPW * 256)])


@functools.partial(
    pl.kernel,
    out_type=jax.ShapeDtypeStruct((NPAD * 256,), jnp.float32),
    mesh=_MESH,
    compiler_params=pltpu.CompilerParams(needs_layout_passes=False),
    scratch_types=[
        pltpu.VMEM((ACCW,), jnp.float32),
        pltpu.VMEM((Q, 256), jnp.float32),
        pltpu.VMEM((Q,), jnp.int32),
        pltpu.VMEM((Q,), jnp.int32),
        pltpu.VMEM((16,), jnp.int32),
        pltpu.SemaphoreType.DMA,
    ],
)
def _sc_prop(xs_hbm, slist, dlist, counts, zrows, out_hbm,
             acc, gbuf, sidx, didx, cntv, sem):
    _prop_body(xs_hbm, slist, dlist, counts, zrows, out_hbm,
               acc, gbuf, sidx, didx, cntv, sem)


def _isq(deg_blk):
    return jax.lax.rsqrt(deg_blk + 1.0)


def _elu(v):
    return jnp.where(v > 0, v, jnp.exp(jnp.minimum(v, 0.0)) - 1.0)


def _k1_body(x_ref, deg_ref, xs_ref):
    xs_ref[...] = x_ref[...] * _isq(deg_ref[...])


def _tc_scale(x_pad, deg_pad):
    return pl.pallas_call(
        _k1_body,
        grid=(GRID,),
        in_specs=[
            pl.BlockSpec((BLK, 256), lambda i: (i, 0)),
            pl.BlockSpec((BLK, 1), lambda i: (i, 0)),
        ],
        out_specs=pl.BlockSpec((BLK, 256), lambda i: (i, 0)),
        out_shape=jax.ShapeDtypeStruct((NPAD, 256), jnp.float32),
    )(x_pad, deg_pad)


def _k2_body(s_ref, xs_ref, deg_ref, w_ref, b_ref, outa_ref, outb_ref):
    isq = _isq(deg_ref[...])
    p = (s_ref[...] + xs_ref[...]) * isq
    h = _elu(jnp.dot(p, w_ref[...], preferred_element_type=jnp.float32)
             + b_ref[...])
    xs1 = h * isq
    outa_ref[...] = xs1[:, :256]
    outb_ref[...] = xs1[:, 256:]


def _tc_layer0(s0, xs0, deg_pad, W0, b0):
    return pl.pallas_call(
        _k2_body,
        grid=(GRID,),
        in_specs=[
            pl.BlockSpec((BLK, 256), lambda i: (i, 0)),
            pl.BlockSpec((BLK, 256), lambda i: (i, 0)),
            pl.BlockSpec((BLK, 1), lambda i: (i, 0)),
            pl.BlockSpec((256, 512), lambda i: (0, 0)),
            pl.BlockSpec((1, 512), lambda i: (0, 0)),
        ],
        out_specs=[
            pl.BlockSpec((BLK, 256), lambda i: (i, 0)),
            pl.BlockSpec((BLK, 256), lambda i: (i, 0)),
        ],
        out_shape=[
            jax.ShapeDtypeStruct((NPAD, 256), jnp.float32),
            jax.ShapeDtypeStruct((NPAD, 256), jnp.float32),
        ],
    )(s0, xs0, deg_pad, W0, b0)


def _k3_body(sa_ref, sb_ref, xa_ref, xb_ref, deg_ref, w1_ref, b1_ref, w2_ref,
             out_ref):
    isq = _isq(deg_ref[...])
    pa = (sa_ref[...] + xa_ref[...]) * isq
    pb = (sb_ref[...] + xb_ref[...]) * isq
    p = jnp.concatenate([pa, pb], axis=1)
    h = _elu(jnp.dot(p, w1_ref[...], preferred_element_type=jnp.float32)
             + b1_ref[...])
    y = jnp.dot(h, w2_ref[...], preferred_element_type=jnp.float32)
    out_ref[...] = y * isq


def _tc_layer1(s1a, s1b, xs1a, xs1b, deg_pad, W1, b1, W2):
    return pl.pallas_call(
        _k3_body,
        grid=(GRID,),
        in_specs=[
            pl.BlockSpec((BLK, 256), lambda i: (i, 0)),
            pl.BlockSpec((BLK, 256), lambda i: (i, 0)),
            pl.BlockSpec((BLK, 256), lambda i: (i, 0)),
            pl.BlockSpec((BLK, 256), lambda i: (i, 0)),
            pl.BlockSpec((BLK, 1), lambda i: (i, 0)),
            pl.BlockSpec((512, 512), lambda i: (0, 0)),
            pl.BlockSpec((1, 512), lambda i: (0, 0)),
            pl.BlockSpec((512, 256), lambda i: (0, 0)),
        ],
        out_specs=pl.BlockSpec((BLK, 256), lambda i: (i, 0)),
        out_shape=jax.ShapeDtypeStruct((NPAD, 256), jnp.float32),
    )(s1a, s1b, xs1a, xs1b, deg_pad, W1, b1, W2)


def _k4_body(s_ref, xs_ref, deg_ref, b_ref, out_ref):
    isq = _isq(deg_ref[...])
    out_ref[...] = (s_ref[...] + xs_ref[...]) * isq + b_ref[...]


def _tc_layer2(s2, xs2, deg_pad, b2):
    return pl.pallas_call(
        _k4_body,
        grid=(GRID,),
        in_specs=[
            pl.BlockSpec((BLK, 256), lambda i: (i, 0)),
            pl.BlockSpec((BLK, 256), lambda i: (i, 0)),
            pl.BlockSpec((BLK, 1), lambda i: (i, 0)),
            pl.BlockSpec((1, 256), lambda i: (0, 0)),
        ],
        out_specs=pl.BlockSpec((BLK, 256), lambda i: (i, 0)),
        out_shape=jax.ShapeDtypeStruct((NPAD, 256), jnp.float32),
    )(s2, xs2, deg_pad, b2)


def kernel(x, edge_index, W0, b0, W1, b1, W2, b2):
    src = edge_index[0]
    dst = edge_index[1]

    slist, dlist, counts, deg = _sc_plan(src, dst)
    deg_pad = deg.reshape(NPAD, 1)

    x_pad = jnp.zeros((NPAD, 256), jnp.float32).at[:N].set(x)
    zrows = jnp.zeros((ACCW,), jnp.float32)

    def prop(vs):
        return _sc_prop(vs, slist, dlist, counts, zrows).reshape(NPAD, 256)

    xs0 = _tc_scale(x_pad, deg_pad)
    s0 = prop(xs0)
    xs1a, xs1b = _tc_layer0(s0, xs0, deg_pad, W0, b0.reshape(1, 512))
    s1a = prop(xs1a)
    s1b = prop(xs1b)
    xs2 = _tc_layer1(s1a, s1b, xs1a, xs1b, deg_pad, W1, b1.reshape(1, 512), W2)
    s2 = prop(xs2)
    out = _tc_layer2(s2, xs2, deg_pad, b2.reshape(1, 256))
    return out[:N]


# speedup vs baseline: 4.3838x; 1.0095x over previous
import functools

import jax
import jax.numpy as jnp
from jax import lax
from jax.experimental import pallas as pl
from jax.experimental.pallas import tpu as pltpu
from jax.experimental.pallas import tpu_sc as plsc

N = 10000
E = 160000
NPAD = 10240
BLK = 256
GRID = NPAD // BLK

NW = 32
NPW = NPAD // NW
Q = 128
C = 2000
NVR = C // 16
CAP = E + Q
STAGE = C + Q + 16

_MESH = plsc.VectorSubcoreMesh(core_axis_name="c", subcore_axis_name="s")


def _plan_body(src_hbm, dst_hbm, slist, dlist, counts, deg_hbm,
               sbuf, dbuf, ss, sd, deg16, degout, cntv):
    wid = lax.axis_index("c") * 16 + lax.axis_index("s")
    lo = wid * NPW
    lanes = lax.iota(jnp.int32, 16)
    zeros16f = jnp.zeros((16,), jnp.float32)
    ones16f = jnp.ones((16,), jnp.float32)

    def _z(t, _):
        deg16[pl.ds(t * 16, 16)] = zeros16f
        return _
    lax.fori_loop(0, NPW, _z, 0)

    def _chunk(ci, carry):
        nflushed, rem = carry
        coff = pl.multiple_of(ci * C, 8)
        pltpu.sync_copy(src_hbm.at[pl.ds(coff, C)], sbuf)
        pltpu.sync_copy(dst_hbm.at[pl.ds(coff, C)], dbuf)

        def _vreg(j, rem):
            s = sbuf[pl.ds(j * 16, 16)]
            d = dbuf[pl.ds(j * 16, 16)]
            dl = d - lo
            m = (dl >= 0) & (dl < NPW)
            dlc = jnp.clip(dl, 0, NPW - 1)
            plsc.store_compressed(ss.at[pl.ds(rem, 16)], s, mask=m)
            plsc.store_compressed(sd.at[pl.ds(rem, 16)], dlc, mask=m)
            plsc.addupdate_scatter(deg16, [lanes * NPW + dlc], ones16f, mask=m)
            return rem + plsc.all_reduce_population_count(m)[0]
        rem = lax.fori_loop(0, NVR, _vreg, rem)

        nq = rem >> 7
        r0 = nq << 7

        def _flush(k, _):
            off = pl.multiple_of(wid * CAP + nflushed + (k << 7), 8)
            k7 = pl.multiple_of(k << 7, 8)
            pltpu.sync_copy(ss.at[pl.ds(k7, Q)], slist.at[pl.ds(off, Q)])
            pltpu.sync_copy(sd.at[pl.ds(k7, Q)], dlist.at[pl.ds(off, Q)])
            return _
        lax.fori_loop(0, nq, _flush, 0)
        for j in range(Q // 16):
            vs = ss[pl.ds(r0 + j * 16, 16)]
            vd = sd[pl.ds(r0 + j * 16, 16)]
            ss[pl.ds(j * 16, 16)] = vs
            sd[pl.ds(j * 16, 16)] = vd
        return nflushed + r0, rem - r0

    nflushed, rem = lax.fori_loop(0, E // C, _chunk, (0, 0))

    for j in range(Q // 16):
        idx = lanes + (j * 16)
        vs = ss[pl.ds(j * 16, 16)]
        vd = sd[pl.ds(j * 16, 16)]
        keep = idx < rem
        ss[pl.ds(j * 16, 16)] = jnp.where(keep, vs, 0)
        sd[pl.ds(j * 16, 16)] = jnp.where(keep, vd, NPW)

    @pl.when(rem > 0)
    def _():
        off = pl.multiple_of(wid * CAP + nflushed, 8)
        pltpu.sync_copy(ss.at[pl.ds(0, Q)], slist.at[pl.ds(off, Q)])
        pltpu.sync_copy(sd.at[pl.ds(0, Q)], dlist.at[pl.ds(off, Q)])

    cntv[...] = jnp.full((16,), nflushed + rem, jnp.int32)
    pltpu.sync_copy(cntv, counts.at[wid])

    def _red(t, _):
        acc = zeros16f
        for l in range(16):
            acc = acc + deg16[pl.ds(l * NPW + t * 16, 16)]
        degout[pl.ds(t * 16, 16)] = acc
        return _
    lax.fori_loop(0, NPW // 16, _red, 0)
    pltpu.sync_copy(degout, deg_hbm.at[pl.ds(pl.multiple_of(wid * NPW, 8), NPW)])


@functools.partial(
    pl.kernel,
    out_type=[
        jax.ShapeDtypeStruct((NW * CAP,), jnp.int32),
        jax.ShapeDtypeStruct((NW * CAP,), jnp.int32),
        jax.ShapeDtypeStruct((NW, 16), jnp.int32),
        jax.ShapeDtypeStruct((NPAD,), jnp.float32),
    ],
    mesh=_MESH,
    compiler_params=pltpu.CompilerParams(needs_layout_passes=False),
    scratch_types=[
        pltpu.VMEM((C,), jnp.int32),
        pltpu.VMEM((C,), jnp.int32),
        pltpu.VMEM((STAGE,), jnp.int32),
        pltpu.VMEM((STAGE,), jnp.int32),
        pltpu.VMEM((16 * NPW,), jnp.float32),
        pltpu.VMEM((NPW,), jnp.float32),
        pltpu.VMEM((16,), jnp.int32),
    ],
)
def _sc_plan(src_hbm, dst_hbm, slist, dlist, counts, deg_hbm,
             sbuf, dbuf, ss, sd, deg16, degout, cntv):
    _plan_body(src_hbm, dst_hbm, slist, dlist, counts, deg_hbm,
               sbuf, dbuf, ss, sd, deg16, degout, cntv)


ACCW = (NPW + 1) * 256


def _prop_body(xs_hbm, slist, dlist, counts, zrows, out_hbm,
               acc, gbuf, sidx, didx, cntv, sem):
    wid = lax.axis_index("c") * 16 + lax.axis_index("s")
    lanes = lax.iota(jnp.int32, 16)
    pltpu.sync_copy(counts.at[wid], cntv)
    cnt = jnp.max(cntv[...])
    nq = (cnt + (Q - 1)) >> 7
    pltpu.sync_copy(zrows, acc)

    def _quantum(k, _):
        off = pl.multiple_of(wid * CAP + (k << 7), 8)
        pltpu.sync_copy(slist.at[pl.ds(off, Q)], sidx)
        pltpu.sync_copy(dlist.at[pl.ds(off, Q)], didx)
        pltpu.async_copy(xs_hbm.at[sidx], gbuf, sem).wait()

        def _vreg(v, _):
            base_v = didx[pl.ds(v * 16, 16)] << 8
            for l in range(16):
                b = pl.multiple_of(base_v[l], 8)
                r = v * 16 + l
                for j in range(16):
                    val = gbuf[r, pl.ds(j * 16, 16)]
                    plsc.addupdate(acc.at[pl.ds(b + j * 16, 16)], val)
            return _
        lax.fori_loop(0, Q // 16, _vreg, 0)
        return _
    lax.fori_loop(0, nq, _quantum, 0)

    pltpu.sync_copy(
        acc.at[pl.ds(0, NPW * 256)],
        out_hbm.at[pl.ds(pl.multiple_of(wid * NPW * 256, 8), NPW * 256)])


@functools.partial(
    pl.kernel,
    out_type=jax.ShapeDtypeStruct((NPAD * 256,), jnp.float32),
    mesh=_MESH,
    compiler_params=pltpu.CompilerParams(needs_layout_passes=False),
    scratch_types=[
        pltpu.VMEM((ACCW,), jnp.float32),
        pltpu.VMEM((Q, 256), jnp.float32),
        pltpu.VMEM((Q,), jnp.int32),
        pltpu.VMEM((Q,), jnp.int32),
        pltpu.VMEM((16,), jnp.int32),
        pltpu.SemaphoreType.DMA,
    ],
)
def _sc_prop(xs_hbm, slist, dlist, counts, zrows, out_hbm,
             acc, gbuf, sidx, didx, cntv, sem):
    _prop_body(xs_hbm, slist, dlist, counts, zrows, out_hbm,
               acc, gbuf, sidx, didx, cntv, sem)


def _isq(deg_blk):
    return jax.lax.rsqrt(deg_blk + 1.0)


def _elu(v):
    return jnp.where(v > 0, v, jnp.exp(jnp.minimum(v, 0.0)) - 1.0)


def _k1_body(x_ref, deg_ref, xs_ref):
    xs_ref[...] = x_ref[...] * _isq(deg_ref[...])


def _tc_scale(x_pad, deg_pad):
    return pl.pallas_call(
        _k1_body,
        grid=(GRID,),
        in_specs=[
            pl.BlockSpec((BLK, 256), lambda i: (i, 0)),
            pl.BlockSpec((BLK, 1), lambda i: (i, 0)),
        ],
        out_specs=pl.BlockSpec((BLK, 256), lambda i: (i, 0)),
        out_shape=jax.ShapeDtypeStruct((NPAD, 256), jnp.float32),
    )(x_pad, deg_pad)


def _k2_body(s_ref, xs_ref, deg_ref, w_ref, b_ref, outa_ref, outb_ref):
    isq = _isq(deg_ref[...])
    p = (s_ref[...] + xs_ref[...]) * isq
    h = _elu(jnp.dot(p, w_ref[...], preferred_element_type=jnp.float32)
             + b_ref[...])
    xs1 = h * isq
    outa_ref[...] = xs1[:, :256]
    outb_ref[...] = xs1[:, 256:]


def _tc_layer0(s0, xs0, deg_pad, W0, b0):
    return pl.pallas_call(
        _k2_body,
        grid=(GRID,),
        in_specs=[
            pl.BlockSpec((BLK, 256), lambda i: (i, 0)),
            pl.BlockSpec((BLK, 256), lambda i: (i, 0)),
            pl.BlockSpec((BLK, 1), lambda i: (i, 0)),
            pl.BlockSpec((256, 512), lambda i: (0, 0)),
            pl.BlockSpec((1, 512), lambda i: (0, 0)),
        ],
        out_specs=[
            pl.BlockSpec((BLK, 256), lambda i: (i, 0)),
            pl.BlockSpec((BLK, 256), lambda i: (i, 0)),
        ],
        out_shape=[
            jax.ShapeDtypeStruct((NPAD, 256), jnp.float32),
            jax.ShapeDtypeStruct((NPAD, 256), jnp.float32),
        ],
    )(s0, xs0, deg_pad, W0, b0)


def _k3_body(sa_ref, sb_ref, xa_ref, xb_ref, deg_ref, w1_ref, b1_ref, w2_ref,
             out_ref):
    isq = _isq(deg_ref[...])
    pa = (sa_ref[...] + xa_ref[...]) * isq
    pb = (sb_ref[...] + xb_ref[...]) * isq
    p = jnp.concatenate([pa, pb], axis=1)
    h = _elu(jnp.dot(p, w1_ref[...], preferred_element_type=jnp.float32)
             + b1_ref[...])
    y = jnp.dot(h, w2_ref[...], preferred_element_type=jnp.float32)
    out_ref[...] = y * isq


def _tc_layer1(s1a, s1b, xs1a, xs1b, deg_pad, W1, b1, W2):
    return pl.pallas_call(
        _k3_body,
        grid=(GRID,),
        in_specs=[
            pl.BlockSpec((BLK, 256), lambda i: (i, 0)),
            pl.BlockSpec((BLK, 256), lambda i: (i, 0)),
            pl.BlockSpec((BLK, 256), lambda i: (i, 0)),
            pl.BlockSpec((BLK, 256), lambda i: (i, 0)),
            pl.BlockSpec((BLK, 1), lambda i: (i, 0)),
            pl.BlockSpec((512, 512), lambda i: (0, 0)),
            pl.BlockSpec((1, 512), lambda i: (0, 0)),
            pl.BlockSpec((512, 256), lambda i: (0, 0)),
        ],
        out_specs=pl.BlockSpec((BLK, 256), lambda i: (i, 0)),
        out_shape=jax.ShapeDtypeStruct((NPAD, 256), jnp.float32),
    )(s1a, s1b, xs1a, xs1b, deg_pad, W1, b1, W2)


def _k4_body(s_ref, xs_ref, deg_ref, b_ref, out_ref):
    isq = _isq(deg_ref[...])
    out_ref[...] = (s_ref[...] + xs_ref[...]) * isq + b_ref[...]


def _tc_layer2(s2, xs2, deg_pad, b2):
    return pl.pallas_call(
        _k4_body,
        grid=(GRID,),
        in_specs=[
            pl.BlockSpec((BLK, 256), lambda i: (i, 0)),
            pl.BlockSpec((BLK, 256), lambda i: (i, 0)),
            pl.BlockSpec((BLK, 1), lambda i: (i, 0)),
            pl.BlockSpec((1, 256), lambda i: (0, 0)),
        ],
        out_specs=pl.BlockSpec((BLK, 256), lambda i: (i, 0)),
        out_shape=jax.ShapeDtypeStruct((NPAD, 256), jnp.float32),
    )(s2, xs2, deg_pad, b2)


def kernel(x, edge_index, W0, b0, W1, b1, W2, b2):
    src = edge_index[0]
    dst = edge_index[1]

    slist, dlist, counts, deg = _sc_plan(src, dst)
    deg_pad = deg.reshape(NPAD, 1)

    x_pad = jnp.zeros((NPAD, 256), jnp.float32).at[:N].set(x)
    zrows = jnp.zeros((ACCW,), jnp.float32)

    def prop(vs):
        return _sc_prop(vs, slist, dlist, counts, zrows).reshape(NPAD, 256)

    xs0 = _tc_scale(x_pad, deg_pad)
    s0 = prop(xs0)
    xs1a, xs1b = _tc_layer0(s0, xs0, deg_pad, W0, b0.reshape(1, 512))
    s1a = prop(xs1a)
    s1b = prop(xs1b)
    xs2 = _tc_layer1(s1a, s1b, xs1a, xs1b, deg_pad, W1, b1.reshape(1, 512), W2)
    s2 = prop(xs2)
    out = _tc_layer2(s2, xs2, deg_pad, b2.reshape(1, 256))
    return out[:N]
